# baseline (speedup 1.0000x reference)
import functools

import jax
import jax.numpy as jnp
from jax import lax
from jax.experimental import pallas as pl
from jax.experimental.pallas import tpu as pltpu
from jax.experimental.pallas import tpu_sc as plsc

RES = 2048
C = 8
N = 1048576
HW = RES * RES

NC = 2
NS = 16
NW = NC * NS
PER_TILE = N // NW
CHUNK = 1024
NJ = CHUNK // 128
NCHUNKS = PER_TILE // CHUNK


def _sc_grid_sample(fx, fy, table):
    mesh = plsc.VectorSubcoreMesh(core_axis_name="c", subcore_axis_name="s")

    @functools.partial(
        pl.kernel,
        mesh=mesh,
        compiler_params=pltpu.CompilerParams(
            needs_layout_passes=False, use_tc_tiling_on_sc=False),
        out_type=jax.ShapeDtypeStruct((N * C,), jnp.float32),
        scratch_types=[
            pltpu.VMEM((CHUNK,), jnp.float32),
            pltpu.VMEM((CHUNK,), jnp.float32),
            pltpu.VMEM((CHUNK,), jnp.int32),
            pltpu.VMEM((CHUNK,), jnp.int32),
            pltpu.VMEM((CHUNK,), jnp.int32),
            pltpu.VMEM((CHUNK,), jnp.int32),
            pltpu.VMEM((CHUNK,), jnp.float32),
            pltpu.VMEM((CHUNK,), jnp.float32),
            pltpu.VMEM((CHUNK,), jnp.float32),
            pltpu.VMEM((CHUNK,), jnp.float32),
            pltpu.VMEM((CHUNK, C), jnp.float32),
            pltpu.VMEM((CHUNK, C), jnp.float32),
            pltpu.VMEM((CHUNK, C), jnp.float32),
            pltpu.VMEM((CHUNK, C), jnp.float32),
            pltpu.VMEM((CHUNK * C,), jnp.float32),
            pltpu.SemaphoreType.DMA,
        ],
    )
    def k(fx_hbm, fy_hbm, tab_hbm, out_hbm,
          fx_v, fy_v, i00, i01, i10, i11, w00, w01, w10, w11,
          r00, r01, r10, r11, out_v, sem):
        wid = lax.axis_index("s") * NC + lax.axis_index("c")
        tile_base = wid * PER_TILE
        lanes = lax.iota(jnp.int32, 16)
        row_off = lax.shift_right_logical(lanes, 3)
        col_idx = jnp.bitwise_and(lanes, 7)

        def chunk_body(g, _):
            base = tile_base + g * CHUNK
            pltpu.sync_copy(fx_hbm.at[pl.ds(base, CHUNK)], fx_v)
            pltpu.sync_copy(fy_hbm.at[pl.ds(base, CHUNK)], fy_v)

            def prep(kk, _):
                o = kk * 16
                ix = fx_v[pl.ds(o, 16)]
                iy = fy_v[pl.ds(o, 16)]
                ix0 = ix.astype(jnp.int32)
                iy0 = iy.astype(jnp.int32)
                wx = ix - ix0.astype(jnp.float32)
                wy = iy - iy0.astype(jnp.float32)
                ix1 = jnp.minimum(ix0 + 1, RES - 1)
                iy1 = jnp.minimum(iy0 + 1, RES - 1)
                y0 = iy0 * RES
                y1 = iy1 * RES
                i00[pl.ds(o, 16)] = y0 + ix0
                i01[pl.ds(o, 16)] = y0 + ix1
                i10[pl.ds(o, 16)] = y1 + ix0
                i11[pl.ds(o, 16)] = y1 + ix1
                ux = 1.0 - wx
                uy = 1.0 - wy
                w00[pl.ds(o, 16)] = ux * uy
                w01[pl.ds(o, 16)] = wx * uy
                w10[pl.ds(o, 16)] = ux * wy
                w11[pl.ds(o, 16)] = wx * wy
                return 0

            lax.fori_loop(0, CHUNK // 16, prep, 0, unroll=2)

            copies = []
            for iv, rv in ((i00, r00), (i01, r01), (i10, r10), (i11, r11)):
                for j in range(NJ):
                    copies.append(pltpu.async_copy(
                        tab_hbm.at[iv.at[pl.ds(j * 128, 128)]],
                        rv.at[pl.ds(j * 128, 128)], sem))
            for cp in copies:
                cp.wait()

            def blend(p, _):
                o = p * 16
                ridx = 2 * p + row_off
                v00 = plsc.load_gather(r00, [ridx, col_idx])
                v01 = plsc.load_gather(r01, [ridx, col_idx])
                v10 = plsc.load_gather(r10, [ridx, col_idx])
                v11 = plsc.load_gather(r11, [ridx, col_idx])
                b00 = plsc.load_gather(w00, [ridx])
                b01 = plsc.load_gather(w01, [ridx])
                b10 = plsc.load_gather(w10, [ridx])
                b11 = plsc.load_gather(w11, [ridx])
                acc = v00 * b00 + v01 * b01 + v10 * b10 + v11 * b11
                out_v[pl.ds(o, 16)] = acc
                return 0

            lax.fori_loop(0, CHUNK // 2, blend, 0, unroll=2)

            pltpu.sync_copy(out_v, out_hbm.at[pl.ds(base * C, CHUNK * C)])
            return 0

        lax.fori_loop(0, NCHUNKS, chunk_body, 0)

    return k(fx, fy, table)


def kernel(x, data):
    xs = jax.nn.sigmoid(x)
    xs = xs * 2.0 - 1.0
    xs = jnp.flip(xs, axis=-1)
    gx = xs[:, 0]
    gy = xs[:, 1]
    fx = jnp.clip((gx + 1.0) * 0.5 * (RES - 1), 0.0, float(RES - 1))
    fy = jnp.clip((gy + 1.0) * 0.5 * (RES - 1), 0.0, float(RES - 1))
    table = data.transpose(1, 2, 0).reshape(HW, C)
    out = _sc_grid_sample(fx, fy, table)
    return out.reshape(N, C)


# speedup vs baseline: 1.3014x; 1.3014x over previous
import functools

import jax
import jax.numpy as jnp
from jax import lax
from jax.experimental import pallas as pl
from jax.experimental.pallas import tpu as pltpu
from jax.experimental.pallas import tpu_sc as plsc

RES = 2048
C = 8
N = 1048576
HW = RES * RES

NC = 2
NS = 16
NW = NC * NS
PER_TILE = N // NW
CHUNK = 1024
NJ = CHUNK // 128
NCHUNKS = PER_TILE // CHUNK


PIX = HW // NW
PCH = 4096
NPCH = PIX // PCH


def _sc_relayout(d2):
    mesh = plsc.VectorSubcoreMesh(core_axis_name="c", subcore_axis_name="s")

    @functools.partial(
        pl.kernel,
        mesh=mesh,
        compiler_params=pltpu.CompilerParams(
            needs_layout_passes=False, use_tc_tiling_on_sc=False),
        out_type=jax.ShapeDtypeStruct((HW * C,), jnp.float32),
        scratch_types=[
            pltpu.VMEM((C, PCH), jnp.float32),
            pltpu.VMEM((PCH * C,), jnp.float32),
        ],
    )
    def k(d_hbm, t_hbm, inb, outb):
        wid = lax.axis_index("s") * NC + lax.axis_index("c")
        tile_base = wid * PIX
        lanes = lax.iota(jnp.int32, 16)
        cidx = jnp.bitwise_and(lanes, 7)
        roff = lax.shift_right_logical(lanes, 3)

        def chunk_body(g, _):
            base = tile_base + g * PCH
            for c in range(C):
                pltpu.sync_copy(d_hbm.at[c, pl.ds(base, PCH)],
                                inb.at[c])

            def interleave(kk, _):
                ridx = 2 * kk + roff
                v = plsc.load_gather(inb, [cidx, ridx])
                outb[pl.ds(kk * 16, 16)] = v
                return 0

            lax.fori_loop(0, PCH // 2, interleave, 0, unroll=8)
            pltpu.sync_copy(outb, t_hbm.at[pl.ds(base * C, PCH * C)])
            return 0

        lax.fori_loop(0, NPCH, chunk_body, 0)

    return k(d2)


def _sc_grid_sample(fx, fy, table):
    mesh = plsc.VectorSubcoreMesh(core_axis_name="c", subcore_axis_name="s")

    @functools.partial(
        pl.kernel,
        mesh=mesh,
        compiler_params=pltpu.CompilerParams(
            needs_layout_passes=False, use_tc_tiling_on_sc=False),
        out_type=jax.ShapeDtypeStruct((N * C,), jnp.float32),
        scratch_types=[
            pltpu.VMEM((CHUNK,), jnp.float32),
            pltpu.VMEM((CHUNK,), jnp.float32),
            pltpu.VMEM((CHUNK,), jnp.int32),
            pltpu.VMEM((CHUNK,), jnp.int32),
            pltpu.VMEM((CHUNK,), jnp.int32),
            pltpu.VMEM((CHUNK,), jnp.int32),
            pltpu.VMEM((CHUNK,), jnp.float32),
            pltpu.VMEM((CHUNK,), jnp.float32),
            pltpu.VMEM((CHUNK,), jnp.float32),
            pltpu.VMEM((CHUNK,), jnp.float32),
            pltpu.VMEM((CHUNK, C), jnp.float32),
            pltpu.VMEM((CHUNK, C), jnp.float32),
            pltpu.VMEM((CHUNK, C), jnp.float32),
            pltpu.VMEM((CHUNK, C), jnp.float32),
            pltpu.VMEM((CHUNK * C,), jnp.float32),
            pltpu.SemaphoreType.DMA,
        ],
    )
    def k(fx_hbm, fy_hbm, tab_hbm, out_hbm,
          fx_v, fy_v, i00, i01, i10, i11, w00, w01, w10, w11,
          r00, r01, r10, r11, out_v, sem):
        wid = lax.axis_index("s") * NC + lax.axis_index("c")
        tile_base = wid * PER_TILE
        lanes = lax.iota(jnp.int32, 16)
        row_off = lax.shift_right_logical(lanes, 3)
        col_idx = jnp.bitwise_and(lanes, 7)

        def chunk_body(g, _):
            base = tile_base + g * CHUNK
            pltpu.sync_copy(fx_hbm.at[pl.ds(base, CHUNK)], fx_v)
            pltpu.sync_copy(fy_hbm.at[pl.ds(base, CHUNK)], fy_v)

            def prep(kk, _):
                o = kk * 16
                ix = fx_v[pl.ds(o, 16)]
                iy = fy_v[pl.ds(o, 16)]
                ix0 = ix.astype(jnp.int32)
                iy0 = iy.astype(jnp.int32)
                wx = ix - ix0.astype(jnp.float32)
                wy = iy - iy0.astype(jnp.float32)
                ix1 = jnp.minimum(ix0 + 1, RES - 1)
                iy1 = jnp.minimum(iy0 + 1, RES - 1)
                y0 = iy0 * RES
                y1 = iy1 * RES
                i00[pl.ds(o, 16)] = y0 + ix0
                i01[pl.ds(o, 16)] = y0 + ix1
                i10[pl.ds(o, 16)] = y1 + ix0
                i11[pl.ds(o, 16)] = y1 + ix1
                ux = 1.0 - wx
                uy = 1.0 - wy
                w00[pl.ds(o, 16)] = ux * uy
                w01[pl.ds(o, 16)] = wx * uy
                w10[pl.ds(o, 16)] = ux * wy
                w11[pl.ds(o, 16)] = wx * wy
                return 0

            lax.fori_loop(0, CHUNK // 16, prep, 0, unroll=2)

            copies = []
            for iv, rv in ((i00, r00), (i01, r01), (i10, r10), (i11, r11)):
                for j in range(NJ):
                    copies.append(pltpu.async_copy(
                        tab_hbm.at[iv.at[pl.ds(j * 128, 128)]],
                        rv.at[pl.ds(j * 128, 128)], sem))
            for cp in copies:
                cp.wait()

            def blend(p, _):
                o = p * 16
                ridx = 2 * p + row_off
                v00 = plsc.load_gather(r00, [ridx, col_idx])
                v01 = plsc.load_gather(r01, [ridx, col_idx])
                v10 = plsc.load_gather(r10, [ridx, col_idx])
                v11 = plsc.load_gather(r11, [ridx, col_idx])
                b00 = plsc.load_gather(w00, [ridx])
                b01 = plsc.load_gather(w01, [ridx])
                b10 = plsc.load_gather(w10, [ridx])
                b11 = plsc.load_gather(w11, [ridx])
                acc = v00 * b00 + v01 * b01 + v10 * b10 + v11 * b11
                out_v[pl.ds(o, 16)] = acc
                return 0

            lax.fori_loop(0, CHUNK // 2, blend, 0, unroll=2)

            pltpu.sync_copy(out_v, out_hbm.at[pl.ds(base * C, CHUNK * C)])
            return 0

        lax.fori_loop(0, NCHUNKS, chunk_body, 0)

    return k(fx, fy, table)


def kernel(x, data):
    xs = jax.nn.sigmoid(x)
    xs = xs * 2.0 - 1.0
    xs = jnp.flip(xs, axis=-1)
    gx = xs[:, 0]
    gy = xs[:, 1]
    fx = jnp.clip((gx + 1.0) * 0.5 * (RES - 1), 0.0, float(RES - 1))
    fy = jnp.clip((gy + 1.0) * 0.5 * (RES - 1), 0.0, float(RES - 1))
    table = _sc_relayout(data.reshape(C, HW)).reshape(HW, C)
    out = _sc_grid_sample(fx, fy, table)
    return out.reshape(N, C)


# speedup vs baseline: 3.3398x; 2.5663x over previous
import functools

import jax
import jax.numpy as jnp
from jax import lax
from jax.experimental import pallas as pl
from jax.experimental.pallas import tpu as pltpu
from jax.experimental.pallas import tpu_sc as plsc

RES = 2048
C = 8
N = 1048576
HW = RES * RES

NC = 2
NS = 16
NW = NC * NS
PER_TILE = N // NW
CHUNK = 1024
NJ = CHUNK // 128
NCHUNKS = PER_TILE // CHUNK


PIX = HW // NW
PCH = 4096
NPCH = PIX // PCH


def _sc_relayout(d2):
    mesh = plsc.VectorSubcoreMesh(core_axis_name="c", subcore_axis_name="s")

    @functools.partial(
        pl.kernel,
        mesh=mesh,
        compiler_params=pltpu.CompilerParams(
            needs_layout_passes=False, use_tc_tiling_on_sc=False),
        out_type=jax.ShapeDtypeStruct((HW * C,), jnp.float32),
        scratch_types=[
            pltpu.VMEM((C, PCH), jnp.float32),
            pltpu.VMEM((PCH * C,), jnp.float32),
        ],
    )
    def k(d_hbm, t_hbm, inb, outb):
        wid = lax.axis_index("s") * NC + lax.axis_index("c")
        tile_base = wid * PIX
        lanes = lax.iota(jnp.int32, 16)
        cidx = jnp.bitwise_and(lanes, 7)
        roff = lax.shift_right_logical(lanes, 3)

        def chunk_body(g, _):
            base = tile_base + g * PCH
            for c in range(C):
                pltpu.sync_copy(d_hbm.at[c, pl.ds(base, PCH)],
                                inb.at[c])

            def interleave(kk, _):
                ridx = 2 * kk + roff
                v = plsc.load_gather(inb, [cidx, ridx])
                outb[pl.ds(kk * 16, 16)] = v
                return 0

            lax.fori_loop(0, PCH // 2, interleave, 0, unroll=8)
            pltpu.sync_copy(outb, t_hbm.at[pl.ds(base * C, PCH * C)])
            return 0

        lax.fori_loop(0, NPCH, chunk_body, 0)

    return k(d2)


def _sc_grid_sample(fx, fy, table):
    mesh = plsc.VectorSubcoreMesh(core_axis_name="c", subcore_axis_name="s")

    @functools.partial(
        pl.kernel,
        mesh=mesh,
        compiler_params=pltpu.CompilerParams(
            needs_layout_passes=False, use_tc_tiling_on_sc=False),
        out_type=jax.ShapeDtypeStruct((N * C,), jnp.float32),
        scratch_types=[
            pltpu.VMEM((CHUNK,), jnp.float32),
            pltpu.VMEM((CHUNK,), jnp.float32),
            pltpu.VMEM((CHUNK,), jnp.int32),
            pltpu.VMEM((CHUNK,), jnp.int32),
            pltpu.VMEM((CHUNK,), jnp.int32),
            pltpu.VMEM((CHUNK,), jnp.int32),
            pltpu.VMEM((CHUNK,), jnp.float32),
            pltpu.VMEM((CHUNK,), jnp.float32),
            pltpu.VMEM((CHUNK,), jnp.float32),
            pltpu.VMEM((CHUNK,), jnp.float32),
            pltpu.VMEM((CHUNK, C), jnp.float32),
            pltpu.VMEM((CHUNK, C), jnp.float32),
            pltpu.VMEM((CHUNK, C), jnp.float32),
            pltpu.VMEM((CHUNK, C), jnp.float32),
            pltpu.VMEM((CHUNK * C,), jnp.float32),
            pltpu.SemaphoreType.DMA,
        ],
    )
    def k(fx_hbm, fy_hbm, tab_hbm, out_hbm,
          fx_v, fy_v, i00, i01, i10, i11, w00, w01, w10, w11,
          r00, r01, r10, r11, out_v, sem):
        wid = lax.axis_index("s") * NC + lax.axis_index("c")
        tile_base = wid * PER_TILE
        lanes = lax.iota(jnp.int32, 16)
        row_off = lax.shift_right_logical(lanes, 3)
        col_idx = jnp.bitwise_and(lanes, 7)

        def chunk_body(g, _):
            base = tile_base + g * CHUNK
            pltpu.sync_copy(fx_hbm.at[pl.ds(base, CHUNK)], fx_v)
            pltpu.sync_copy(fy_hbm.at[pl.ds(base, CHUNK)], fy_v)

            def prep(kk, _):
                o = kk * 16
                ix = fx_v[pl.ds(o, 16)]
                iy = fy_v[pl.ds(o, 16)]
                ix0 = ix.astype(jnp.int32)
                iy0 = iy.astype(jnp.int32)
                wx = ix - ix0.astype(jnp.float32)
                wy = iy - iy0.astype(jnp.float32)
                ix1 = jnp.minimum(ix0 + 1, RES - 1)
                iy1 = jnp.minimum(iy0 + 1, RES - 1)
                y0 = iy0 * RES
                y1 = iy1 * RES
                i00[pl.ds(o, 16)] = y0 + ix0
                i01[pl.ds(o, 16)] = y0 + ix1
                i10[pl.ds(o, 16)] = y1 + ix0
                i11[pl.ds(o, 16)] = y1 + ix1
                ux = 1.0 - wx
                uy = 1.0 - wy
                w00[pl.ds(o, 16)] = ux * uy
                w01[pl.ds(o, 16)] = wx * uy
                w10[pl.ds(o, 16)] = ux * wy
                w11[pl.ds(o, 16)] = wx * wy
                return 0

            lax.fori_loop(0, CHUNK // 16, prep, 0, unroll=2)

            copies = []
            for iv, rv in ((i00, r00), (i01, r01), (i10, r10), (i11, r11)):
                for j in range(NJ):
                    copies.append(pltpu.async_copy(
                        tab_hbm.at[iv.at[pl.ds(j * 128, 128)]],
                        rv.at[pl.ds(j * 128, 128)], sem))
            for cp in copies:
                cp.wait()

            def blend(p, _):
                o = p * 16
                ridx = 2 * p + row_off
                v00 = plsc.load_gather(r00, [ridx, col_idx])
                v01 = plsc.load_gather(r01, [ridx, col_idx])
                v10 = plsc.load_gather(r10, [ridx, col_idx])
                v11 = plsc.load_gather(r11, [ridx, col_idx])
                b00 = plsc.load_gather(w00, [ridx])
                b01 = plsc.load_gather(w01, [ridx])
                b10 = plsc.load_gather(w10, [ridx])
                b11 = plsc.load_gather(w11, [ridx])
                acc = v00 * b00 + v01 * b01 + v10 * b10 + v11 * b11
                out_v[pl.ds(o, 16)] = acc
                return 0

            lax.fori_loop(0, CHUNK // 2, blend, 0, unroll=2)

            pltpu.sync_copy(out_v, out_hbm.at[pl.ds(base * C, CHUNK * C)])
            return 0

        lax.fori_loop(0, NCHUNKS, chunk_body, 0)

    return k(fx, fy, table)


def kernel(x, data):
    xs = jax.nn.sigmoid(x)
    xs = xs * 2.0 - 1.0
    gx = xs[:, 1]
    gy = xs[:, 0]
    fx = jnp.clip((gx + 1.0) * 0.5 * (RES - 1), 0.0, float(RES - 1))
    fy = jnp.clip((gy + 1.0) * 0.5 * (RES - 1), 0.0, float(RES - 1))
    table = _sc_relayout(data.reshape(C, HW)).reshape(HW, C)
    out = _sc_grid_sample(fx, fy, table)
    return out.reshape(N, C)


# speedup vs baseline: 3.8315x; 1.1472x over previous
import functools

import jax
import jax.numpy as jnp
from jax import lax
from jax.experimental import pallas as pl
from jax.experimental.pallas import tpu as pltpu
from jax.experimental.pallas import tpu_sc as plsc

RES = 2048
C = 8
N = 1048576
HW = RES * RES

NC = 2
NS = 16
NW = NC * NS
PER_TILE = N // NW
CHUNK = 1024
NJ = CHUNK // 128
NCHUNKS = PER_TILE // CHUNK


PIX = HW // NW
PCH = 4096
NPCH = PIX // PCH


def _sc_relayout(d2):
    mesh = plsc.VectorSubcoreMesh(core_axis_name="c", subcore_axis_name="s")

    @functools.partial(
        pl.kernel,
        mesh=mesh,
        compiler_params=pltpu.CompilerParams(
            needs_layout_passes=False, use_tc_tiling_on_sc=False),
        out_type=jax.ShapeDtypeStruct((HW * C,), jnp.float32),
        scratch_types=[
            pltpu.VMEM((C, PCH), jnp.float32),
            pltpu.VMEM((C, PCH), jnp.float32),
            pltpu.VMEM((PCH * C,), jnp.float32),
            pltpu.VMEM((PCH * C,), jnp.float32),
            pltpu.SemaphoreType.DMA,
            pltpu.SemaphoreType.DMA,
        ],
    )
    def k(d_hbm, t_hbm, inb0, inb1, outb0, outb1, isem, osem):
        wid = lax.axis_index("s") * NC + lax.axis_index("c")
        tile_base = wid * PIX
        lanes = lax.iota(jnp.int32, 16)
        cidx = jnp.bitwise_and(lanes, 7)
        roff = lax.shift_right_logical(lanes, 3)
        inb = (inb0, inb1)
        outb = (outb0, outb1)

        def fire_in(g, buf):
            base = tile_base + g * PCH
            return [pltpu.async_copy(d_hbm.at[c, pl.ds(base, PCH)],
                                     buf.at[c], isem)
                    for c in range(C)]

        in_cp = fire_in(0, inb[0])
        out_cp = [None, None]
        for g in range(NPCH):
            cur = g % 2
            if g + 1 < NPCH:
                nxt_cp = fire_in(g + 1, inb[(g + 1) % 2])
            for cp in in_cp:
                cp.wait()
            if g + 1 < NPCH:
                in_cp = nxt_cp
            if out_cp[cur] is not None:
                out_cp[cur].wait()
            src = inb[cur]
            dst = outb[cur]

            def interleave(kk, _, src=src, dst=dst):
                ridx = 2 * kk + roff
                v = plsc.load_gather(src, [cidx, ridx])
                dst[pl.ds(kk * 16, 16)] = v
                return 0

            lax.fori_loop(0, PCH // 2, interleave, 0, unroll=8)
            base = tile_base + g * PCH
            out_cp[cur] = pltpu.async_copy(
                dst, t_hbm.at[pl.ds(base * C, PCH * C)], osem)
        for cp in out_cp:
            if cp is not None:
                cp.wait()

    return k(d2)


def _sc_grid_sample(fx, fy, table):
    mesh = plsc.VectorSubcoreMesh(core_axis_name="c", subcore_axis_name="s")

    @functools.partial(
        pl.kernel,
        mesh=mesh,
        compiler_params=pltpu.CompilerParams(
            needs_layout_passes=False, use_tc_tiling_on_sc=False),
        out_type=jax.ShapeDtypeStruct((N, C), jnp.float32),
        scratch_types=[
            pltpu.VMEM((CHUNK,), jnp.float32),
            pltpu.VMEM((CHUNK,), jnp.float32),
            pltpu.VMEM((CHUNK,), jnp.int32),
            pltpu.VMEM((CHUNK,), jnp.int32),
            pltpu.VMEM((CHUNK,), jnp.int32),
            pltpu.VMEM((CHUNK,), jnp.int32),
            pltpu.VMEM((CHUNK,), jnp.float32),
            pltpu.VMEM((CHUNK,), jnp.float32),
            pltpu.VMEM((CHUNK,), jnp.float32),
            pltpu.VMEM((CHUNK,), jnp.float32),
            pltpu.VMEM((CHUNK, C), jnp.float32),
            pltpu.VMEM((CHUNK, C), jnp.float32),
            pltpu.VMEM((CHUNK, C), jnp.float32),
            pltpu.VMEM((CHUNK, C), jnp.float32),
            pltpu.VMEM((CHUNK, C), jnp.float32),
            pltpu.SemaphoreType.DMA,
        ],
    )
    def k(fx_hbm, fy_hbm, tab_hbm, out_hbm,
          fx_v, fy_v, i00, i01, i10, i11, w00, w01, w10, w11,
          r00, r01, r10, r11, out_v, sem):
        wid = lax.axis_index("s") * NC + lax.axis_index("c")
        tile_base = wid * PER_TILE
        lanes = lax.iota(jnp.int32, 16)
        row_off = lax.shift_right_logical(lanes, 3)
        col_idx = jnp.bitwise_and(lanes, 7)

        def chunk_body(g, _):
            base = tile_base + g * CHUNK
            pltpu.sync_copy(fx_hbm.at[pl.ds(base, CHUNK)], fx_v)
            pltpu.sync_copy(fy_hbm.at[pl.ds(base, CHUNK)], fy_v)

            def prep(kk, _):
                o = kk * 16
                ix = fx_v[pl.ds(o, 16)]
                iy = fy_v[pl.ds(o, 16)]
                ix0 = ix.astype(jnp.int32)
                iy0 = iy.astype(jnp.int32)
                wx = ix - ix0.astype(jnp.float32)
                wy = iy - iy0.astype(jnp.float32)
                ix1 = jnp.minimum(ix0 + 1, RES - 1)
                iy1 = jnp.minimum(iy0 + 1, RES - 1)
                y0 = iy0 * RES
                y1 = iy1 * RES
                i00[pl.ds(o, 16)] = y0 + ix0
                i01[pl.ds(o, 16)] = y0 + ix1
                i10[pl.ds(o, 16)] = y1 + ix0
                i11[pl.ds(o, 16)] = y1 + ix1
                ux = 1.0 - wx
                uy = 1.0 - wy
                w00[pl.ds(o, 16)] = ux * uy
                w01[pl.ds(o, 16)] = wx * uy
                w10[pl.ds(o, 16)] = ux * wy
                w11[pl.ds(o, 16)] = wx * wy
                return 0

            lax.fori_loop(0, CHUNK // 16, prep, 0, unroll=2)

            copies = []
            for iv, rv in ((i00, r00), (i01, r01), (i10, r10), (i11, r11)):
                for j in range(NJ):
                    copies.append(pltpu.async_copy(
                        tab_hbm.at[iv.at[pl.ds(j * 128, 128)]],
                        rv.at[pl.ds(j * 128, 128)], sem))
            for cp in copies:
                cp.wait()

            def blend(p, _):
                o = p * 16
                ridx = 2 * p + row_off
                v00 = plsc.load_gather(r00, [ridx, col_idx])
                v01 = plsc.load_gather(r01, [ridx, col_idx])
                v10 = plsc.load_gather(r10, [ridx, col_idx])
                v11 = plsc.load_gather(r11, [ridx, col_idx])
                b00 = plsc.load_gather(w00, [ridx])
                b01 = plsc.load_gather(w01, [ridx])
                b10 = plsc.load_gather(w10, [ridx])
                b11 = plsc.load_gather(w11, [ridx])
                acc = v00 * b00 + v01 * b01 + v10 * b10 + v11 * b11
                plsc.store_scatter(out_v, [ridx, col_idx], acc)
                return 0

            lax.fori_loop(0, CHUNK // 2, blend, 0, unroll=2)

            pltpu.sync_copy(out_v, out_hbm.at[pl.ds(base, CHUNK)])
            return 0

        lax.fori_loop(0, NCHUNKS, chunk_body, 0)

    return k(fx, fy, table)


def kernel(x, data):
    xs = jax.nn.sigmoid(x)
    xs = xs * 2.0 - 1.0
    gx = xs[:, 1]
    gy = xs[:, 0]
    fx = jnp.clip((gx + 1.0) * 0.5 * (RES - 1), 0.0, float(RES - 1))
    fy = jnp.clip((gy + 1.0) * 0.5 * (RES - 1), 0.0, float(RES - 1))
    table = _sc_relayout(data.reshape(C, HW)).reshape(HW, C)
    return _sc_grid_sample(fx, fy, table)


# speedup vs baseline: 4.0447x; 1.0556x over previous
import functools

import jax
import jax.numpy as jnp
from jax import lax
from jax.experimental import pallas as pl
from jax.experimental.pallas import tpu as pltpu
from jax.experimental.pallas import tpu_sc as plsc

RES = 2048
C = 8
N = 1048576
HW = RES * RES

NC = 2
NS = 16
NW = NC * NS
PER_TILE = N // NW
CHUNK = 1024
NJ = CHUNK // 128
NCHUNKS = PER_TILE // CHUNK


PIX = HW // NW
PCH = 4096
NPCH = PIX // PCH


def _sc_relayout(d2):
    mesh = plsc.VectorSubcoreMesh(core_axis_name="c", subcore_axis_name="s")

    @functools.partial(
        pl.kernel,
        mesh=mesh,
        compiler_params=pltpu.CompilerParams(
            needs_layout_passes=False, use_tc_tiling_on_sc=False),
        out_type=jax.ShapeDtypeStruct((HW * C,), jnp.float32),
        scratch_types=[
            pltpu.VMEM((C, PCH), jnp.float32),
            pltpu.VMEM((C, PCH), jnp.float32),
            pltpu.VMEM((PCH * C,), jnp.float32),
            pltpu.VMEM((PCH * C,), jnp.float32),
            pltpu.SemaphoreType.DMA,
            pltpu.SemaphoreType.DMA,
        ],
    )
    def k(d_hbm, t_hbm, inb0, inb1, outb0, outb1, isem, osem):
        wid = lax.axis_index("s") * NC + lax.axis_index("c")
        tile_base = wid * PIX
        lanes = lax.iota(jnp.int32, 16)
        cidx = jnp.bitwise_and(lanes, 7)
        roff = lax.shift_right_logical(lanes, 3)
        inb = (inb0, inb1)
        outb = (outb0, outb1)

        def fire_in(g, buf):
            base = tile_base + g * PCH
            return [pltpu.async_copy(d_hbm.at[c, pl.ds(base, PCH)],
                                     buf.at[c], isem)
                    for c in range(C)]

        in_cp = fire_in(0, inb[0])
        out_cp = [None, None]
        for g in range(NPCH):
            cur = g % 2
            if g + 1 < NPCH:
                nxt_cp = fire_in(g + 1, inb[(g + 1) % 2])
            for cp in in_cp:
                cp.wait()
            if g + 1 < NPCH:
                in_cp = nxt_cp
            if out_cp[cur] is not None:
                out_cp[cur].wait()
            src = inb[cur]
            dst = outb[cur]

            def interleave(kk, rid, src=src, dst=dst):
                v = plsc.load_gather(src, [cidx, rid])
                dst[pl.ds(kk * 16, 16)] = v
                return rid + 2

            lax.fori_loop(0, PCH // 2, interleave, roff, unroll=8)
            base = tile_base + g * PCH
            out_cp[cur] = pltpu.async_copy(
                dst, t_hbm.at[pl.ds(base * C, PCH * C)], osem)
        for cp in out_cp:
            if cp is not None:
                cp.wait()

    return k(d2)


def _sc_grid_sample(fx, fy, table):
    mesh = plsc.VectorSubcoreMesh(core_axis_name="c", subcore_axis_name="s")

    slot_scratch = [
        pltpu.VMEM((CHUNK,), jnp.float32),
        pltpu.VMEM((CHUNK,), jnp.float32),
        pltpu.VMEM((CHUNK,), jnp.int32),
        pltpu.VMEM((CHUNK,), jnp.int32),
        pltpu.VMEM((CHUNK,), jnp.int32),
        pltpu.VMEM((CHUNK,), jnp.int32),
        pltpu.VMEM((CHUNK, C), jnp.float32),
        pltpu.VMEM((CHUNK, C), jnp.float32),
        pltpu.VMEM((CHUNK, C), jnp.float32),
        pltpu.VMEM((CHUNK, C), jnp.float32),
        pltpu.VMEM((CHUNK, C), jnp.float32),
    ]

    @functools.partial(
        pl.kernel,
        mesh=mesh,
        compiler_params=pltpu.CompilerParams(
            needs_layout_passes=False, use_tc_tiling_on_sc=False),
        out_type=jax.ShapeDtypeStruct((N, C), jnp.float32),
        scratch_types=slot_scratch + slot_scratch + [
            pltpu.SemaphoreType.DMA,
            pltpu.SemaphoreType.DMA,
            pltpu.SemaphoreType.DMA,
            pltpu.SemaphoreType.DMA,
            pltpu.SemaphoreType.DMA,
            pltpu.SemaphoreType.DMA,
        ],
    )
    def k(fx_hbm, fy_hbm, tab_hbm, out_hbm, *sc):
        nslot = len(slot_scratch)
        slots = (sc[:nslot], sc[nslot:2 * nslot])
        isem = (sc[2 * nslot], sc[2 * nslot + 1])
        gsem = (sc[2 * nslot + 2], sc[2 * nslot + 3])
        osem = (sc[2 * nslot + 4], sc[2 * nslot + 5])

        wid = lax.axis_index("s") * NC + lax.axis_index("c")
        tile_base = wid * PER_TILE
        lanes = lax.iota(jnp.int32, 16)
        row_off = lax.shift_right_logical(lanes, 3)
        col_idx = jnp.bitwise_and(lanes, 7)

        def fire_in(g, s):
            base = tile_base + g * CHUNK
            fxv, fyv = slots[s][0], slots[s][1]
            pltpu.async_copy(fx_hbm.at[pl.ds(base, CHUNK)], fxv, isem[s])
            pltpu.async_copy(fy_hbm.at[pl.ds(base, CHUNK)], fyv, isem[s])

        def wait_in(s):
            fxv, fyv = slots[s][0], slots[s][1]
            pltpu.make_async_copy(
                fx_hbm.at[pl.ds(0, CHUNK)], fxv, isem[s]).wait()
            pltpu.make_async_copy(
                fy_hbm.at[pl.ds(0, CHUNK)], fyv, isem[s]).wait()

        def prep(s):
            fxv, fyv = slots[s][0], slots[s][1]
            idxs = slots[s][2:6]

            def body(kk, _):
                o = kk * 16
                ix = fxv[pl.ds(o, 16)]
                iy = fyv[pl.ds(o, 16)]
                ix0 = ix.astype(jnp.int32)
                iy0 = iy.astype(jnp.int32)
                ix1 = jnp.minimum(ix0 + 1, RES - 1)
                iy1 = jnp.minimum(iy0 + 1, RES - 1)
                y0 = iy0 * RES
                y1 = iy1 * RES
                idxs[0][pl.ds(o, 16)] = y0 + ix0
                idxs[1][pl.ds(o, 16)] = y0 + ix1
                idxs[2][pl.ds(o, 16)] = y1 + ix0
                idxs[3][pl.ds(o, 16)] = y1 + ix1
                return 0

            lax.fori_loop(0, CHUNK // 16, body, 0, unroll=2)

        def fire_gather(s):
            idxs = slots[s][2:6]
            rows = slots[s][6:10]

            def body(j, _):
                for t in range(4):
                    pltpu.async_copy(
                        tab_hbm.at[idxs[t].at[pl.ds(j * 128, 128)]],
                        rows[t].at[pl.ds(j * 128, 128)], gsem[s])
                return 0

            lax.fori_loop(0, NJ, body, 0)

        def wait_gather(s):
            rows = slots[s][6:10]
            for t in range(4):
                pltpu.make_async_copy(
                    tab_hbm.at[pl.ds(0, CHUNK)], rows[t], gsem[s]).wait()

        def blend(s):
            fxv, fyv = slots[s][0], slots[s][1]
            r00, r01, r10, r11, out_v = slots[s][6:11]

            def body(p, rid):
                v00 = plsc.load_gather(r00, [rid, col_idx])
                v01 = plsc.load_gather(r01, [rid, col_idx])
                v10 = plsc.load_gather(r10, [rid, col_idx])
                v11 = plsc.load_gather(r11, [rid, col_idx])
                bfx = plsc.load_gather(fxv, [rid])
                bfy = plsc.load_gather(fyv, [rid])
                wx = bfx - bfx.astype(jnp.int32).astype(jnp.float32)
                wy = bfy - bfy.astype(jnp.int32).astype(jnp.float32)
                ux = 1.0 - wx
                uy = 1.0 - wy
                acc = (v00 * (ux * uy) + v01 * (wx * uy)
                       + v10 * (ux * wy) + v11 * (wx * wy))
                plsc.store_scatter(out_v, [rid, col_idx], acc)
                return rid + 2

            lax.fori_loop(0, CHUNK // 2, body, row_off, unroll=2)

        def fire_out(g, s):
            base = tile_base + g * CHUNK
            pltpu.async_copy(slots[s][10], out_hbm.at[pl.ds(base, CHUNK)],
                             osem[s])

        def wait_out(s):
            pltpu.make_async_copy(
                slots[s][10], out_hbm.at[pl.ds(0, CHUNK)], osem[s]).wait()

        fire_in(0, 0)
        for g in range(NCHUNKS):
            s = g % 2
            wait_in(s)
            prep(s)
            fire_gather(s)
            if g >= 1:
                ps = (g - 1) % 2
                wait_gather(ps)
                if g >= 3:
                    wait_out(ps)
                blend(ps)
                fire_out(g - 1, ps)
            if g + 1 < NCHUNKS:
                fire_in(g + 1, (g + 1) % 2)
        ls = (NCHUNKS - 1) % 2
        wait_gather(ls)
        wait_out(ls)
        blend(ls)
        fire_out(NCHUNKS - 1, ls)
        wait_out(0)
        wait_out(1)

    return k(fx, fy, table)


def kernel(x, data):
    xs = jax.nn.sigmoid(x)
    xs = xs * 2.0 - 1.0
    gx = xs[:, 1]
    gy = xs[:, 0]
    fx = jnp.clip((gx + 1.0) * 0.5 * (RES - 1), 0.0, float(RES - 1))
    fy = jnp.clip((gy + 1.0) * 0.5 * (RES - 1), 0.0, float(RES - 1))
    table = _sc_relayout(data.reshape(C, HW)).reshape(HW, C)
    return _sc_grid_sample(fx, fy, table)
